# baseline (speedup 1.0000x reference)
import jax
import jax.numpy as jnp
from jax import lax
from jax.experimental import pallas as pl
from jax.experimental.pallas import tpu as pltpu
from jax.experimental.pallas import tpu_sc as plsc

B = 1024
P = 12
S = B * P
NW = 32
SPW = S // NW
C = 8
NCH = SPW // C
IDXW = SPW * 17

_OFF_POK = 0
_OFF_SA = SPW
_OFF_SI = SPW * 4
_OFF_SM = SPW * 6
_OFF_ABIL = SPW * 10
_OFF_MV = SPW * 11
_OFF_ITEM = SPW * 15
_OFF_PREP = SPW * 16

_GATHERS = (
    (_OFF_POK, 1, 0, 0),
    (_OFF_SA, 3, 1, 1),
    (_OFF_SI, 2, 2, 2),
    (_OFF_SM, 4, 3, 3),
    (_OFF_ABIL, 1, 1, 4),
    (_OFF_MV, 4, 3, 5),
    (_OFF_ITEM, 1, 2, 6),
    (_OFF_PREP, 1, 3, 7),
)
_BUFSHAPES = ((C, 384), (3 * C, 128), (2 * C, 128), (4 * C, 256),
              (C, 128), (4 * C, 256), (C, 128), (C, 256))


def _sc_body(idx_hbm, pok_h, ab_h, it_h, mv_h,
             o_pok, o_sa, o_si, o_sm, o_abil, o_mv, o_item, o_prep,
             idx_v, bufs0, bufs1, sems):
    tables = (pok_h, ab_h, it_h, mv_h)
    bufsets = (bufs0, bufs1)
    outs = (o_pok, o_sa, o_si, o_sm, o_abil, o_mv, o_item, o_prep)
    wid = lax.axis_index("s") * 2 + lax.axis_index("c")
    base = wid * SPW
    pltpu.sync_copy(idx_hbm.at[wid], idx_v)
    gsem = (sems[0], sems[1])
    osem = (sems[2], sems[3])

    def issue_gathers(i, par):
        bufs = bufsets[par]
        for n, (off, k, t, b) in enumerate(_GATHERS):
            io = pl.multiple_of(off + i * (k * C), 8)
            pltpu.async_copy(
                tables[t].at[idx_v.at[pl.ds(io, k * C)]], bufs[b], gsem[par])

    def wait_gathers(par):
        bufs = bufsets[par]
        for (off, k, t, b) in _GATHERS:
            pltpu.make_async_copy(
                tables[t].at[pl.ds(0, k * C)], bufs[b], gsem[par]).wait()

    def issue_outs(i, par):
        bufs = bufsets[par]
        s0 = base + i * C
        for (off, k, t, b) in _GATHERS:
            oo = pl.multiple_of(k * s0, 8)
            pltpu.async_copy(bufs[b], outs[b].at[pl.ds(oo, k * C)], osem[par])

    def wait_outs(par):
        bufs = bufsets[par]
        for (off, k, t, b) in _GATHERS:
            pltpu.make_async_copy(
                bufs[b], outs[b].at[pl.ds(0, k * C)], osem[par]).wait()

    issue_gathers(0, 0)

    def step(i, carry):
        for par in (0, 1):
            j = i + par
            wait_gathers(par)
            issue_outs(j, par)
            nxt = 1 - par
            nj = j + 1

            @pl.when(nj < NCH)
            def _():
                @pl.when(nj >= 2)
                def _():
                    wait_outs(nxt)
                issue_gathers(nj, nxt)
        return carry

    lax.fori_loop(0, NCH // 2, lambda it, c: step(2 * it, c), 0, unroll=False)
    wait_outs(0)
    wait_outs(1)


_Q = 8


def _asm_body(pok, sa, si, sm, abil, mv, item, prep, raw, out):
    rawv = raw[...]
    for q in range(_Q):
        rows = pl.ds(q * C, C)
        for p in range(12):
            seg = jnp.concatenate([
                pok[p, q][:, :291],
                sa[p, q, 0][:, :51], sa[p, q, 1][:, :51], sa[p, q, 2][:, :51],
                si[p, q, 0][:, :51], si[p, q, 1][:, :51],
                sm[p, q, 0][:, :154], sm[p, q, 1][:, :154],
                sm[p, q, 2][:, :154], sm[p, q, 3][:, :154],
                rawv[q * C:(q + 1) * C, p, :],
                abil[p, q][:, :51],
                mv[p, q, 0][:, :154], mv[p, q, 1][:, :154],
                mv[p, q, 2][:, :154], mv[p, q, 3][:, :154],
                item[p, q][:, :51], prep[p, q][:, :154]], axis=-1)
            out[rows, p * 2371:(p + 1) * 2371] = seg


def _assemble(o, raw):
    nc = B // C
    f = pl.pallas_call(
        _asm_body,
        grid=(nc // _Q,),
        in_specs=[
            pl.BlockSpec((P, _Q, C, 384), lambda i: (0, i, 0, 0)),
            pl.BlockSpec((P, _Q, 3, C, 128), lambda i: (0, i, 0, 0, 0)),
            pl.BlockSpec((P, _Q, 2, C, 128), lambda i: (0, i, 0, 0, 0)),
            pl.BlockSpec((P, _Q, 4, C, 256), lambda i: (0, i, 0, 0, 0)),
            pl.BlockSpec((P, _Q, C, 128), lambda i: (0, i, 0, 0)),
            pl.BlockSpec((P, _Q, 4, C, 256), lambda i: (0, i, 0, 0, 0)),
            pl.BlockSpec((P, _Q, C, 128), lambda i: (0, i, 0, 0)),
            pl.BlockSpec((P, _Q, C, 256), lambda i: (0, i, 0, 0)),
            pl.BlockSpec((_Q * C, P, 337), lambda i: (i, 0, 0)),
        ],
        out_specs=pl.BlockSpec((_Q * C, P * 2371), lambda i: (i, 0)),
        out_shape=jax.ShapeDtypeStruct((B, P * 2371), jnp.float32),
    )
    return f(o[0].reshape(P, nc, C, 384),
             o[1].reshape(P, nc, 3, C, 128),
             o[2].reshape(P, nc, 2, C, 128),
             o[3].reshape(P, nc, 4, C, 256),
             o[4].reshape(P, nc, C, 128),
             o[5].reshape(P, nc, 4, C, 256),
             o[6].reshape(P, nc, C, 128),
             o[7].reshape(P, nc, C, 256),
             raw)


@jax.jit
def _run(idx, pok_h, ab_h, it_h, mv_h):
    mesh = plsc.VectorSubcoreMesh(core_axis_name="c", subcore_axis_name="s")
    f = pl.kernel(
        _sc_body,
        out_type=(
            jax.ShapeDtypeStruct((S, 384), jnp.float32),
            jax.ShapeDtypeStruct((3 * S, 128), jnp.float32),
            jax.ShapeDtypeStruct((2 * S, 128), jnp.float32),
            jax.ShapeDtypeStruct((4 * S, 256), jnp.float32),
            jax.ShapeDtypeStruct((S, 128), jnp.float32),
            jax.ShapeDtypeStruct((4 * S, 256), jnp.float32),
            jax.ShapeDtypeStruct((S, 128), jnp.float32),
            jax.ShapeDtypeStruct((S, 256), jnp.float32),
        ),
        mesh=mesh,
        scratch_types=[
            pltpu.VMEM((IDXW,), jnp.int32),
            tuple(pltpu.VMEM(s, jnp.float32) for s in _BUFSHAPES),
            tuple(pltpu.VMEM(s, jnp.float32) for s in _BUFSHAPES),
            (pltpu.SemaphoreType.DMA, pltpu.SemaphoreType.DMA,
             pltpu.SemaphoreType.DMA, pltpu.SemaphoreType.DMA),
        ],
    )
    return f(idx, pok_h, ab_h, it_h, mv_h)


def kernel(pokemon_ids, species_ability_ids, species_item_ids,
           species_move_ids, ability_ids, move_ids, item_ids,
           preparing_move_ids, raw_features, pokemon_static, pokemon_learn,
           ability_static, ability_learn, item_static, item_learn,
           move_static, move_learn):
    def wm(x):
        x = x.reshape(B, P, -1).astype(jnp.int32)
        k = x.shape[2]
        x = x.transpose(1, 0, 2).reshape(P, B // C, C, k)
        return x.transpose(0, 1, 3, 2).reshape(NW, -1)
    idx = jnp.concatenate([
        wm(pokemon_ids), wm(species_ability_ids), wm(species_item_ids),
        wm(species_move_ids), wm(ability_ids), wm(move_ids),
        wm(item_ids), wm(preparing_move_ids)], axis=1)
    def pad_to(x, w):
        return jnp.pad(x, ((0, 0), (0, w - x.shape[1])))
    pok_h = pad_to(jnp.concatenate([pokemon_static, pokemon_learn], axis=1), 384)
    ab_h = pad_to(jnp.concatenate([ability_static, ability_learn], axis=1), 128)
    it_h = pad_to(jnp.concatenate([item_static, item_learn], axis=1), 128)
    mv_h = pad_to(jnp.concatenate([move_static, move_learn], axis=1), 256)
    o = _run(idx, pok_h, ab_h, it_h, mv_h)
    return _assemble(o, raw_features.astype(jnp.float32))


# speedup vs baseline: 1.1262x; 1.1262x over previous
import jax
import jax.numpy as jnp
from jax import lax
from jax.experimental import pallas as pl
from jax.experimental.pallas import tpu as pltpu
from jax.experimental.pallas import tpu_sc as plsc

B = 1024
P = 12
S = B * P
NW = 32
SPW = S // NW
C = 16
NCH = SPW // C
IDXW = SPW * 17

_OFF_POK = 0
_OFF_SA = SPW
_OFF_SI = SPW * 4
_OFF_SM = SPW * 6
_OFF_ABIL = SPW * 10
_OFF_MV = SPW * 11
_OFF_ITEM = SPW * 15
_OFF_PREP = SPW * 16

_GATHERS = (
    (_OFF_POK, 1, 0, 0),
    (_OFF_SA, 3, 1, 1),
    (_OFF_SI, 2, 2, 2),
    (_OFF_SM, 4, 3, 3),
    (_OFF_ABIL, 1, 1, 4),
    (_OFF_MV, 4, 3, 5),
    (_OFF_ITEM, 1, 2, 6),
    (_OFF_PREP, 1, 3, 7),
)
_BUFSHAPES = ((C, 384), (3 * C, 128), (2 * C, 128), (4 * C, 256),
              (C, 128), (4 * C, 256), (C, 128), (C, 256))


def _sc_body(idx_hbm, pok_h, ab_h, it_h, mv_h,
             o_pok, o_sa, o_si, o_sm, o_abil, o_mv, o_item, o_prep,
             idx_v, bufs0, bufs1, sems):
    tables = (pok_h, ab_h, it_h, mv_h)
    bufsets = (bufs0, bufs1)
    outs = (o_pok, o_sa, o_si, o_sm, o_abil, o_mv, o_item, o_prep)
    wid = lax.axis_index("s") * 2 + lax.axis_index("c")
    base = wid * SPW
    pltpu.sync_copy(idx_hbm.at[wid], idx_v)
    gsem = (sems[0], sems[1])
    osem = (sems[2], sems[3])

    def issue_gathers(i, par):
        bufs = bufsets[par]
        for n, (off, k, t, b) in enumerate(_GATHERS):
            io = pl.multiple_of(off + i * (k * C), 8)
            pltpu.async_copy(
                tables[t].at[idx_v.at[pl.ds(io, k * C)]], bufs[b], gsem[par])

    def wait_gathers(par):
        bufs = bufsets[par]
        for (off, k, t, b) in _GATHERS:
            pltpu.make_async_copy(
                tables[t].at[pl.ds(0, k * C)], bufs[b], gsem[par]).wait()

    def issue_outs(i, par):
        bufs = bufsets[par]
        s0 = base + i * C
        for (off, k, t, b) in _GATHERS:
            oo = pl.multiple_of(k * s0, 8)
            pltpu.async_copy(bufs[b], outs[b].at[pl.ds(oo, k * C)], osem[par])

    def wait_outs(par):
        bufs = bufsets[par]
        for (off, k, t, b) in _GATHERS:
            pltpu.make_async_copy(
                bufs[b], outs[b].at[pl.ds(0, k * C)], osem[par]).wait()

    issue_gathers(0, 0)

    def step(i, carry):
        for par in (0, 1):
            j = i + par
            wait_gathers(par)
            issue_outs(j, par)
            nxt = 1 - par
            nj = j + 1

            @pl.when(nj < NCH)
            def _():
                @pl.when(nj >= 2)
                def _():
                    wait_outs(nxt)
                issue_gathers(nj, nxt)
        return carry

    lax.fori_loop(0, NCH // 2, lambda it, c: step(2 * it, c), 0, unroll=False)
    wait_outs(0)
    wait_outs(1)


_Q = 2


def _asm_body(pok, sa, si, sm, abil, mv, item, prep, raw, out):
    rawv = raw[...]
    for q in range(_Q):
        rows = pl.ds(q * C, C)
        for p in range(12):
            seg = jnp.concatenate([
                pok[p, q][:, :291],
                sa[p, q, 0][:, :51], sa[p, q, 1][:, :51], sa[p, q, 2][:, :51],
                si[p, q, 0][:, :51], si[p, q, 1][:, :51],
                sm[p, q, 0][:, :154], sm[p, q, 1][:, :154],
                sm[p, q, 2][:, :154], sm[p, q, 3][:, :154],
                rawv[q * C:(q + 1) * C, p, :],
                abil[p, q][:, :51],
                mv[p, q, 0][:, :154], mv[p, q, 1][:, :154],
                mv[p, q, 2][:, :154], mv[p, q, 3][:, :154],
                item[p, q][:, :51], prep[p, q][:, :154]], axis=-1)
            out[rows, p * 2371:(p + 1) * 2371] = seg


def _assemble(o, raw):
    nc = B // C
    f = pl.pallas_call(
        _asm_body,
        grid=(nc // _Q,),
        in_specs=[
            pl.BlockSpec((P, _Q, C, 384), lambda i: (0, i, 0, 0)),
            pl.BlockSpec((P, _Q, 3, C, 128), lambda i: (0, i, 0, 0, 0)),
            pl.BlockSpec((P, _Q, 2, C, 128), lambda i: (0, i, 0, 0, 0)),
            pl.BlockSpec((P, _Q, 4, C, 256), lambda i: (0, i, 0, 0, 0)),
            pl.BlockSpec((P, _Q, C, 128), lambda i: (0, i, 0, 0)),
            pl.BlockSpec((P, _Q, 4, C, 256), lambda i: (0, i, 0, 0, 0)),
            pl.BlockSpec((P, _Q, C, 128), lambda i: (0, i, 0, 0)),
            pl.BlockSpec((P, _Q, C, 256), lambda i: (0, i, 0, 0)),
            pl.BlockSpec((_Q * C, P, 337), lambda i: (i, 0, 0)),
        ],
        out_specs=pl.BlockSpec((_Q * C, P * 2371), lambda i: (i, 0)),
        out_shape=jax.ShapeDtypeStruct((B, P * 2371), jnp.float32),
    )
    return f(o[0].reshape(P, nc, C, 384),
             o[1].reshape(P, nc, 3, C, 128),
             o[2].reshape(P, nc, 2, C, 128),
             o[3].reshape(P, nc, 4, C, 256),
             o[4].reshape(P, nc, C, 128),
             o[5].reshape(P, nc, 4, C, 256),
             o[6].reshape(P, nc, C, 128),
             o[7].reshape(P, nc, C, 256),
             raw)


@jax.jit
def _run(idx, pok_h, ab_h, it_h, mv_h):
    mesh = plsc.VectorSubcoreMesh(core_axis_name="c", subcore_axis_name="s")
    f = pl.kernel(
        _sc_body,
        out_type=(
            jax.ShapeDtypeStruct((S, 384), jnp.float32),
            jax.ShapeDtypeStruct((3 * S, 128), jnp.float32),
            jax.ShapeDtypeStruct((2 * S, 128), jnp.float32),
            jax.ShapeDtypeStruct((4 * S, 256), jnp.float32),
            jax.ShapeDtypeStruct((S, 128), jnp.float32),
            jax.ShapeDtypeStruct((4 * S, 256), jnp.float32),
            jax.ShapeDtypeStruct((S, 128), jnp.float32),
            jax.ShapeDtypeStruct((S, 256), jnp.float32),
        ),
        mesh=mesh,
        scratch_types=[
            pltpu.VMEM((IDXW,), jnp.int32),
            tuple(pltpu.VMEM(s, jnp.float32) for s in _BUFSHAPES),
            tuple(pltpu.VMEM(s, jnp.float32) for s in _BUFSHAPES),
            (pltpu.SemaphoreType.DMA, pltpu.SemaphoreType.DMA,
             pltpu.SemaphoreType.DMA, pltpu.SemaphoreType.DMA),
        ],
    )
    return f(idx, pok_h, ab_h, it_h, mv_h)


def kernel(pokemon_ids, species_ability_ids, species_item_ids,
           species_move_ids, ability_ids, move_ids, item_ids,
           preparing_move_ids, raw_features, pokemon_static, pokemon_learn,
           ability_static, ability_learn, item_static, item_learn,
           move_static, move_learn):
    def wm(x):
        x = x.reshape(B, P, -1).astype(jnp.int32)
        k = x.shape[2]
        x = x.transpose(1, 0, 2).reshape(P, B // C, C, k)
        return x.transpose(0, 1, 3, 2).reshape(NW, -1)
    idx = jnp.concatenate([
        wm(pokemon_ids), wm(species_ability_ids), wm(species_item_ids),
        wm(species_move_ids), wm(ability_ids), wm(move_ids),
        wm(item_ids), wm(preparing_move_ids)], axis=1)
    def pad_to(x, w):
        return jnp.pad(x, ((0, 0), (0, w - x.shape[1])))
    pok_h = pad_to(jnp.concatenate([pokemon_static, pokemon_learn], axis=1), 384)
    ab_h = pad_to(jnp.concatenate([ability_static, ability_learn], axis=1), 128)
    it_h = pad_to(jnp.concatenate([item_static, item_learn], axis=1), 128)
    mv_h = pad_to(jnp.concatenate([move_static, move_learn], axis=1), 256)
    o = _run(idx, pok_h, ab_h, it_h, mv_h)
    return _assemble(o, raw_features.astype(jnp.float32))


# speedup vs baseline: 1.3035x; 1.1574x over previous
import jax
import jax.numpy as jnp
from jax import lax
from jax.experimental import pallas as pl
from jax.experimental.pallas import tpu as pltpu
from jax.experimental.pallas import tpu_sc as plsc

B = 1024
P = 12
S = B * P
NW = 32
SPW = S // NW
C = 16
NCH = SPW // C
IDXW = SPW * 17

_OFF_POK = 0
_OFF_SA = SPW
_OFF_SI = SPW * 4
_OFF_SM = SPW * 6
_OFF_ABIL = SPW * 10
_OFF_MV = SPW * 11
_OFF_ITEM = SPW * 15
_OFF_PREP = SPW * 16

_GATHERS = (
    (_OFF_POK, 1, 0, 0),
    (_OFF_SA, 3, 1, 1),
    (_OFF_SI, 2, 2, 2),
    (_OFF_SM, 4, 3, 3),
    (_OFF_ABIL, 1, 1, 4),
    (_OFF_MV, 4, 3, 5),
    (_OFF_ITEM, 1, 2, 6),
    (_OFF_PREP, 1, 3, 7),
)
_BUFSHAPES = ((C, 384), (3 * C, 128), (2 * C, 128), (4 * C, 256),
              (C, 128), (4 * C, 256), (C, 128), (C, 256))


def _sc_body(idx_hbm, pok_h, ab_h, it_h, mv_h,
             o_pok, o_sa, o_si, o_sm, o_abil, o_mv, o_item, o_prep,
             idx_v, bufs0, bufs1, sems):
    tables = (pok_h, ab_h, it_h, mv_h)
    bufsets = (bufs0, bufs1)
    outs = (o_pok, o_sa, o_si, o_sm, o_abil, o_mv, o_item, o_prep)
    wid = lax.axis_index("s") * 2 + lax.axis_index("c")
    base = wid * SPW
    pltpu.sync_copy(idx_hbm.at[wid], idx_v)
    gsem = (sems[0], sems[1])
    osem = (sems[2], sems[3])

    def issue_gathers(i, par):
        bufs = bufsets[par]
        for n, (off, k, t, b) in enumerate(_GATHERS):
            io = pl.multiple_of(off + i * (k * C), 8)
            pltpu.async_copy(
                tables[t].at[idx_v.at[pl.ds(io, k * C)]], bufs[b], gsem[par])

    def wait_gathers(par):
        bufs = bufsets[par]
        for (off, k, t, b) in _GATHERS:
            pltpu.make_async_copy(
                tables[t].at[pl.ds(0, k * C)], bufs[b], gsem[par]).wait()

    def issue_outs(i, par):
        bufs = bufsets[par]
        s0 = base + i * C
        for (off, k, t, b) in _GATHERS:
            oo = pl.multiple_of(k * s0, 8)
            pltpu.async_copy(bufs[b], outs[b].at[pl.ds(oo, k * C)], osem[par])

    def wait_outs(par):
        bufs = bufsets[par]
        for (off, k, t, b) in _GATHERS:
            pltpu.make_async_copy(
                bufs[b], outs[b].at[pl.ds(0, k * C)], osem[par]).wait()

    issue_gathers(0, 0)

    def step(i, carry):
        for par in (0, 1):
            j = i + par
            wait_gathers(par)
            issue_outs(j, par)
            nxt = 1 - par
            nj = j + 1

            @pl.when(nj < NCH)
            def _():
                @pl.when(nj >= 2)
                def _():
                    wait_outs(nxt)
                issue_gathers(nj, nxt)
        return carry

    lax.fori_loop(0, NCH // 2, lambda it, c: step(2 * it, c), 0, unroll=False)
    wait_outs(0)
    wait_outs(1)


_QB = 128
_NQ = _QB // C


def _asm_body(pok, sa, si, sm, abil, mv, item, prep, raw, out):
    rawv = raw[...]

    def branch(p):
        def f():
            segs = []
            for q in range(_NQ):
                segs.append(jnp.concatenate([
                    pok[0, q][:, :291],
                    sa[0, q, 0][:, :51], sa[0, q, 1][:, :51],
                    sa[0, q, 2][:, :51],
                    si[0, q, 0][:, :51], si[0, q, 1][:, :51],
                    sm[0, q, 0][:, :154], sm[0, q, 1][:, :154],
                    sm[0, q, 2][:, :154], sm[0, q, 3][:, :154],
                    rawv[q * C:(q + 1) * C, p, :],
                    abil[0, q][:, :51],
                    mv[0, q, 0][:, :154], mv[0, q, 1][:, :154],
                    mv[0, q, 2][:, :154], mv[0, q, 3][:, :154],
                    item[0, q][:, :51], prep[0, q][:, :154]], axis=-1))
            blk = jnp.concatenate(segs, axis=0)
            out[p * 2371:(p + 1) * 2371, :] = blk.T
        return f

    lax.switch(pl.program_id(1), [branch(p) for p in range(12)])


def _assemble(o, raw):
    nc = B // C
    f = pl.pallas_call(
        _asm_body,
        grid=(B // _QB, P),
        in_specs=[
            pl.BlockSpec((1, _NQ, C, 384), lambda i, p: (p, i, 0, 0)),
            pl.BlockSpec((1, _NQ, 3, C, 128), lambda i, p: (p, i, 0, 0, 0)),
            pl.BlockSpec((1, _NQ, 2, C, 128), lambda i, p: (p, i, 0, 0, 0)),
            pl.BlockSpec((1, _NQ, 4, C, 256), lambda i, p: (p, i, 0, 0, 0)),
            pl.BlockSpec((1, _NQ, C, 128), lambda i, p: (p, i, 0, 0)),
            pl.BlockSpec((1, _NQ, 4, C, 256), lambda i, p: (p, i, 0, 0, 0)),
            pl.BlockSpec((1, _NQ, C, 128), lambda i, p: (p, i, 0, 0)),
            pl.BlockSpec((1, _NQ, C, 256), lambda i, p: (p, i, 0, 0)),
            pl.BlockSpec((_QB, P, 337), lambda i, p: (i, 0, 0)),
        ],
        out_specs=pl.BlockSpec((P * 2371, _QB), lambda i, p: (0, i)),
        out_shape=jax.ShapeDtypeStruct((P * 2371, B), jnp.float32),
        compiler_params=pltpu.CompilerParams(
            vmem_limit_bytes=56 * 1024 * 1024),
    )
    return f(o[0].reshape(P, nc, C, 384),
             o[1].reshape(P, nc, 3, C, 128),
             o[2].reshape(P, nc, 2, C, 128),
             o[3].reshape(P, nc, 4, C, 256),
             o[4].reshape(P, nc, C, 128),
             o[5].reshape(P, nc, 4, C, 256),
             o[6].reshape(P, nc, C, 128),
             o[7].reshape(P, nc, C, 256),
             raw)


@jax.jit
def _run(idx, pok_h, ab_h, it_h, mv_h):
    mesh = plsc.VectorSubcoreMesh(core_axis_name="c", subcore_axis_name="s")
    f = pl.kernel(
        _sc_body,
        out_type=(
            jax.ShapeDtypeStruct((S, 384), jnp.float32),
            jax.ShapeDtypeStruct((3 * S, 128), jnp.float32),
            jax.ShapeDtypeStruct((2 * S, 128), jnp.float32),
            jax.ShapeDtypeStruct((4 * S, 256), jnp.float32),
            jax.ShapeDtypeStruct((S, 128), jnp.float32),
            jax.ShapeDtypeStruct((4 * S, 256), jnp.float32),
            jax.ShapeDtypeStruct((S, 128), jnp.float32),
            jax.ShapeDtypeStruct((S, 256), jnp.float32),
        ),
        mesh=mesh,
        scratch_types=[
            pltpu.VMEM((IDXW,), jnp.int32),
            tuple(pltpu.VMEM(s, jnp.float32) for s in _BUFSHAPES),
            tuple(pltpu.VMEM(s, jnp.float32) for s in _BUFSHAPES),
            (pltpu.SemaphoreType.DMA, pltpu.SemaphoreType.DMA,
             pltpu.SemaphoreType.DMA, pltpu.SemaphoreType.DMA),
        ],
    )
    return f(idx, pok_h, ab_h, it_h, mv_h)


def kernel(pokemon_ids, species_ability_ids, species_item_ids,
           species_move_ids, ability_ids, move_ids, item_ids,
           preparing_move_ids, raw_features, pokemon_static, pokemon_learn,
           ability_static, ability_learn, item_static, item_learn,
           move_static, move_learn):
    def wm(x):
        x = x.reshape(B, P, -1).astype(jnp.int32)
        k = x.shape[2]
        x = x.transpose(1, 0, 2).reshape(P, B // C, C, k)
        return x.transpose(0, 1, 3, 2).reshape(NW, -1)
    idx = jnp.concatenate([
        wm(pokemon_ids), wm(species_ability_ids), wm(species_item_ids),
        wm(species_move_ids), wm(ability_ids), wm(move_ids),
        wm(item_ids), wm(preparing_move_ids)], axis=1)
    def pad_to(x, w):
        return jnp.pad(x, ((0, 0), (0, w - x.shape[1])))
    pok_h = pad_to(jnp.concatenate([pokemon_static, pokemon_learn], axis=1), 384)
    ab_h = pad_to(jnp.concatenate([ability_static, ability_learn], axis=1), 128)
    it_h = pad_to(jnp.concatenate([item_static, item_learn], axis=1), 128)
    mv_h = pad_to(jnp.concatenate([move_static, move_learn], axis=1), 256)
    o = _run(idx, pok_h, ab_h, it_h, mv_h)
    return _assemble(o, raw_features.astype(jnp.float32)).T
